# baseline (device time: 16492 ns/iter reference)
import jax
import jax.numpy as jnp
from jax import lax
from jax.experimental import pallas as pl
from jax.experimental.pallas import tpu as pltpu

QROWS = 128


def kernel(dy, W):
    m, k = dy.shape
    d = W.shape[0]

    def body(dy_ref, w_ref, out_ref, w_bf_ref, send1, recv1, send2, recv2,
             sem_s1, sem_r1, sems_s2, sems_r2):
        my_x = lax.axis_index("x")
        my_y = lax.axis_index("y")
        my_z = lax.axis_index("z")
        base = 2 * my_x + my_y
        q_me = (1 - my_z) * base + my_z * (3 - base)
        q_zn = 3 - q_me
        q_xn = lax.bitwise_xor(q_me, 2)
        q_yn = lax.bitwise_xor(q_me, 1)
        x_nbr = (1 - my_x, my_y, my_z)
        y_nbr = (my_x, 1 - my_y, my_z)
        z_nbr = (my_x, my_y, 1 - my_z)

        barrier = pltpu.get_barrier_semaphore()
        for nbr in (x_nbr, y_nbr, z_nbr):
            pl.semaphore_signal(
                barrier, inc=1, device_id=nbr,
                device_id_type=pl.DeviceIdType.MESH,
            )

        w_bf_ref[...] = w_ref[...].astype(jnp.bfloat16)

        a1 = dy_ref[pl.ds(q_zn * QROWS, QROWS), :].astype(jnp.bfloat16)
        p_zn = lax.dot_general(
            a1, w_bf_ref[...], (((1,), (1,)), ((), ())),
            preferred_element_type=jnp.float32,
        )
        send1[...] = p_zn.astype(jnp.bfloat16)

        pl.semaphore_wait(barrier, 3)

        rdma1 = pltpu.make_async_remote_copy(
            src_ref=send1, dst_ref=recv1,
            send_sem=sem_s1, recv_sem=sem_r1,
            device_id=z_nbr, device_id_type=pl.DeviceIdType.MESH,
        )
        rdma1.start()

        a2 = dy_ref[pl.ds(q_me * QROWS, QROWS), :].astype(jnp.bfloat16)
        p_me = lax.dot_general(
            a2, w_bf_ref[...], (((1,), (1,)), ((), ())),
            preferred_element_type=jnp.float32,
        )

        rdma1.wait_recv()
        r_f32 = p_me + recv1[...].astype(jnp.float32)
        send2[...] = r_f32.astype(jnp.bfloat16)

        rdmas2 = []
        for slot, nbr in ((0, x_nbr), (1, y_nbr), (2, z_nbr)):
            rdma = pltpu.make_async_remote_copy(
                src_ref=send2, dst_ref=recv2.at[slot],
                send_sem=sems_s2.at[slot], recv_sem=sems_r2.at[slot],
                device_id=nbr, device_id_type=pl.DeviceIdType.MESH,
            )
            rdma.start()
            rdmas2.append(rdma)

        out_ref[pl.ds(q_me * QROWS, QROWS), :] = r_f32

        for slot, q in ((0, q_xn), (1, q_yn), (2, q_zn)):
            rdmas2[slot].wait_recv()
            out_ref[pl.ds(q * QROWS, QROWS), :] = recv2[slot].astype(jnp.float32)

        rdma1.wait_send()
        for rdma in rdmas2:
            rdma.wait_send()

    return pl.pallas_call(
        body,
        out_shape=jax.ShapeDtypeStruct((m, d), jnp.float32),
        in_specs=[
            pl.BlockSpec(memory_space=pltpu.VMEM),
            pl.BlockSpec(memory_space=pltpu.VMEM),
        ],
        out_specs=pl.BlockSpec(memory_space=pltpu.VMEM),
        scratch_shapes=[
            pltpu.VMEM((d, k), jnp.bfloat16),
            pltpu.VMEM((QROWS, d), jnp.bfloat16),
            pltpu.VMEM((QROWS, d), jnp.bfloat16),
            pltpu.VMEM((QROWS, d), jnp.bfloat16),
            pltpu.VMEM((3, QROWS, d), jnp.bfloat16),
            pltpu.SemaphoreType.DMA,
            pltpu.SemaphoreType.DMA,
            pltpu.SemaphoreType.DMA((3,)),
            pltpu.SemaphoreType.DMA((3,)),
        ],
        compiler_params=pltpu.CompilerParams(collective_id=0),
    )(dy, W)


# device time: 16038 ns/iter; 1.0283x vs baseline; 1.0283x over previous
import jax
import jax.numpy as jnp
from jax import lax
from jax.experimental import pallas as pl
from jax.experimental.pallas import tpu as pltpu

NCHUNK = 8


def kernel(dy, W):
    m, k = dy.shape
    d = W.shape[0]
    rows = m // NCHUNK

    def body(dy_ref, w_ref, out_ref, w_bf_ref, send_buf, recv_buf,
             send_sems, recv_sems):
        my_x = lax.axis_index("x")
        my_y = lax.axis_index("y")
        my_z = lax.axis_index("z")
        nbr = (my_x, my_y, 1 - my_z)

        barrier = pltpu.get_barrier_semaphore()
        pl.semaphore_signal(
            barrier, inc=1, device_id=nbr,
            device_id_type=pl.DeviceIdType.MESH,
        )

        w_bf_ref[...] = w_ref[...].astype(jnp.bfloat16)

        rdmas = []
        for c in range(NCHUNK):
            a = dy_ref[pl.ds(c * rows, rows), :].astype(jnp.bfloat16)
            p = lax.dot_general(
                a, w_bf_ref[...], (((1,), (1,)), ((), ())),
                preferred_element_type=jnp.float32,
            )
            send_buf[c] = p.astype(jnp.bfloat16)
            if c == 0:
                pl.semaphore_wait(barrier, 1)
            rdma = pltpu.make_async_remote_copy(
                src_ref=send_buf.at[c],
                dst_ref=recv_buf.at[c],
                send_sem=send_sems.at[c],
                recv_sem=recv_sems.at[c],
                device_id=nbr,
                device_id_type=pl.DeviceIdType.MESH,
            )
            rdma.start()
            rdmas.append(rdma)

        for c in range(NCHUNK):
            rdmas[c].wait_recv()
            out_ref[pl.ds(c * rows, rows), :] = (
                send_buf[c].astype(jnp.float32)
                + recv_buf[c].astype(jnp.float32)
            )

        for c in range(NCHUNK):
            rdmas[c].wait_send()

    return pl.pallas_call(
        body,
        out_shape=jax.ShapeDtypeStruct((m, d), jnp.float32),
        in_specs=[
            pl.BlockSpec(memory_space=pltpu.VMEM),
            pl.BlockSpec(memory_space=pltpu.VMEM),
        ],
        out_specs=pl.BlockSpec(memory_space=pltpu.VMEM),
        scratch_shapes=[
            pltpu.VMEM((d, k), jnp.bfloat16),
            pltpu.VMEM((NCHUNK, rows, d), jnp.bfloat16),
            pltpu.VMEM((NCHUNK, rows, d), jnp.bfloat16),
            pltpu.SemaphoreType.DMA((NCHUNK,)),
            pltpu.SemaphoreType.DMA((NCHUNK,)),
        ],
        compiler_params=pltpu.CompilerParams(collective_id=0),
    )(dy, W)


# device time: 15914 ns/iter; 1.0363x vs baseline; 1.0078x over previous
import jax
import jax.numpy as jnp
from jax import lax
from jax.experimental import pallas as pl
from jax.experimental.pallas import tpu as pltpu

NCHUNK = 4


def kernel(dy, W):
    m, k = dy.shape
    d = W.shape[0]
    rows = m // NCHUNK

    def body(dy_ref, w_ref, out_ref, w_bf_ref, send_buf, recv_buf,
             send_sems, recv_sems):
        my_x = lax.axis_index("x")
        my_y = lax.axis_index("y")
        my_z = lax.axis_index("z")
        nbr = (my_x, my_y, 1 - my_z)

        barrier = pltpu.get_barrier_semaphore()
        pl.semaphore_signal(
            barrier, inc=1, device_id=nbr,
            device_id_type=pl.DeviceIdType.MESH,
        )

        w_bf_ref[...] = w_ref[...].astype(jnp.bfloat16)

        rdmas = []
        for c in range(NCHUNK):
            a = dy_ref[pl.ds(c * rows, rows), :].astype(jnp.bfloat16)
            p = lax.dot_general(
                a, w_bf_ref[...], (((1,), (1,)), ((), ())),
                preferred_element_type=jnp.float32,
            )
            send_buf[c] = p.astype(jnp.bfloat16)
            if c == 0:
                pl.semaphore_wait(barrier, 1)
            rdma = pltpu.make_async_remote_copy(
                src_ref=send_buf.at[c],
                dst_ref=recv_buf.at[c],
                send_sem=send_sems.at[c],
                recv_sem=recv_sems.at[c],
                device_id=nbr,
                device_id_type=pl.DeviceIdType.MESH,
            )
            rdma.start()
            rdmas.append(rdma)

        for c in range(NCHUNK):
            rdmas[c].wait_recv()
            out_ref[pl.ds(c * rows, rows), :] = (
                send_buf[c].astype(jnp.float32)
                + recv_buf[c].astype(jnp.float32)
            )

        for c in range(NCHUNK):
            rdmas[c].wait_send()

    return pl.pallas_call(
        body,
        out_shape=jax.ShapeDtypeStruct((m, d), jnp.float32),
        in_specs=[
            pl.BlockSpec(memory_space=pltpu.VMEM),
            pl.BlockSpec(memory_space=pltpu.VMEM),
        ],
        out_specs=pl.BlockSpec(memory_space=pltpu.VMEM),
        scratch_shapes=[
            pltpu.VMEM((d, k), jnp.bfloat16),
            pltpu.VMEM((NCHUNK, rows, d), jnp.bfloat16),
            pltpu.VMEM((NCHUNK, rows, d), jnp.bfloat16),
            pltpu.SemaphoreType.DMA((NCHUNK,)),
            pltpu.SemaphoreType.DMA((NCHUNK,)),
        ],
        compiler_params=pltpu.CompilerParams(collective_id=0),
    )(dy, W)
